# baseline (speedup 1.0000x reference)
import functools

import numpy as np

import jax
import jax.numpy as jnp
from jax.experimental import pallas as pl

F = 51
FP = 64
K = 128
BINS = 2
G = 2
T = 2048
R = T // G
CG = 2 * G * FP
KG = G * K


def _band_maps(ksize, ctr):
    tap = np.full((3, CG, G), ksize, np.int32)
    cin = np.zeros((3, CG, G), np.int32)
    for o in (-1, 0, 1):
        for row in range(CG):
            part, rem = divmod(row, G * FP)
            r, c = divmod(rem, FP)
            if c >= F:
                continue
            for s in range(G):
                t = G * o + r - s + ctr
                if 0 <= t < ksize:
                    tap[o + 1, row, s] = t
                    cin[o + 1, row, s] = part * F + c
    return tap, cin


def _band_maps_y(ksize, ctr):
    tap = np.full((3, KG, G), ksize, np.int32)
    cin = np.zeros((3, KG, G), np.int32)
    for o in (-1, 0, 1):
        for row in range(KG):
            r, kk = divmod(row, K)
            for s in range(G):
                t = G * o + r - s + ctr
                if 0 <= t < ksize:
                    tap[o + 1, row, s] = t
                    cin[o + 1, row, s] = 2 * F + kk
    return tap, cin


_W0_TAP, _W0_CIN = _band_maps(3, 1)
_W1X_TAP, _W1X_CIN = _band_maps(5, 2)
_W1Y_TAP, _W1Y_CIN = _band_maps_y(5, 2)

_WHX_CIN = np.full((CG, G), 2 * F + K, np.int32)
_WHY_CIN = np.full((KG, G), 2 * F + K, np.int32)
for _row in range(CG):
    _part, _rem = divmod(_row, G * FP)
    _r, _c = divmod(_rem, FP)
    if _c < F:
        _WHX_CIN[_row, _r] = _part * F + _c
for _row in range(KG):
    _r, _kk = divmod(_row, K)
    _WHY_CIN[_row, _r] = 2 * F + _kk


def _fused_kernel(prev_ref, cur_ref, next_ref,
                  w0b_ref, w1xb_ref, w1yb_ref, whx_ref, why_ref,
                  b0g_ref, b1g_ref, bhg_ref,
                  out_ref, *, n_groups):
    i = pl.program_id(0)
    f32 = jnp.float32
    xe = jnp.concatenate(
        [prev_ref[R - 2:, :], cur_ref[...], next_ref[:2, :]], axis=0)
    ge = i * R - 2 + jax.lax.broadcasted_iota(jnp.int32, (R + 4, 1), 0)
    xe = jnp.where((ge >= 0) & (ge < n_groups), xe, 0.0)
    xc = jnp.concatenate(
        [jnp.minimum(xe, 0.0), jnp.maximum(xe, 0.0)], axis=1)

    acc0 = jnp.broadcast_to(b0g_ref[...], (R + 2, KG)).astype(f32)
    for o in range(3):
        acc0 = acc0 + jnp.dot(xc[o:o + R + 2], w0b_ref[o],
                              preferred_element_type=f32)
    y0 = jnp.maximum(acc0, 0.0)
    y0 = jnp.where((ge[1:R + 3] >= 0) & (ge[1:R + 3] < n_groups), y0, 0.0)
    y0 = y0.astype(xe.dtype)

    acc1 = jnp.broadcast_to(b1g_ref[...], (R, KG)).astype(f32)
    for o in range(3):
        acc1 = acc1 + jnp.dot(xc[1 + o:1 + o + R], w1xb_ref[o],
                              preferred_element_type=f32)
        acc1 = acc1 + jnp.dot(y0[o:o + R], w1yb_ref[o],
                              preferred_element_type=f32)
    y1 = jnp.maximum(acc1, 0.0).astype(xe.dtype)

    z = (jnp.dot(xc[2:2 + R], whx_ref[...], preferred_element_type=f32)
         + jnp.dot(y1, why_ref[...], preferred_element_type=f32)
         + bhg_ref[...])
    out_ref[...] = jax.nn.softplus(z)


def kernel(inputs, W0, b0, W1, b1, Wh, bh):
    x = inputs[0]
    n, f = x.shape
    nb = n // T
    ng = n // G
    cdt = jnp.bfloat16

    xg = jnp.pad(x.astype(cdt), ((0, 0), (0, FP - f))).reshape(ng, G * FP)

    w0ext = jnp.pad(W0, ((0, 1), (0, 0), (0, 0)))
    w1ext = jnp.pad(W1, ((0, 1), (0, 0), (0, 0)))
    whext = jnp.pad(Wh, ((0, 1), (0, 0)))
    w0b = w0ext[_W0_TAP, _W0_CIN].reshape(3, CG, KG).astype(cdt)
    w1xb = w1ext[_W1X_TAP, _W1X_CIN].reshape(3, CG, KG).astype(cdt)
    w1yb = w1ext[_W1Y_TAP, _W1Y_CIN].reshape(3, KG, KG).astype(cdt)
    whx = whext[_WHX_CIN].reshape(CG, G * BINS).astype(cdt)
    why = whext[_WHY_CIN].reshape(KG, G * BINS).astype(cdt)
    b0g = jnp.tile(b0, G).reshape(1, KG)
    b1g = jnp.tile(b1, G).reshape(1, KG)
    bhg = jnp.tile(bh, G).reshape(1, G * BINS)

    full = lambda shape: pl.BlockSpec(shape, lambda i: (0,) * len(shape))
    out = pl.pallas_call(
        functools.partial(_fused_kernel, n_groups=ng),
        grid=(nb,),
        in_specs=[
            pl.BlockSpec((R, G * FP),
                         lambda i: (jnp.maximum(i - 1, 0), 0)),
            pl.BlockSpec((R, G * FP), lambda i: (i, 0)),
            pl.BlockSpec((R, G * FP),
                         lambda i: (jnp.minimum(i + 1, nb - 1), 0)),
            full((3, CG, KG)), full((3, CG, KG)), full((3, KG, KG)),
            full((CG, G * BINS)), full((KG, G * BINS)),
            full((1, KG)), full((1, KG)), full((1, G * BINS)),
        ],
        out_specs=pl.BlockSpec((R, G * BINS), lambda i: (i, 0)),
        out_shape=jax.ShapeDtypeStruct((ng, G * BINS), jnp.float32),
    )(xg, xg, xg, w0b, w1xb, w1yb, whx, why, b0g, b1g, bhg)
    return out.reshape(n, BINS)[None]


# speedup vs baseline: 1.0277x; 1.0277x over previous
import functools

import numpy as np

import jax
import jax.numpy as jnp
from jax.experimental import pallas as pl

F = 51
FP = 64
K = 128
BINS = 2
G = 2
T = 4096
R = T // G
CG = 2 * G * FP
KG = G * K


def _band_maps(ksize, ctr):
    tap = np.full((3, CG, G), ksize, np.int32)
    cin = np.zeros((3, CG, G), np.int32)
    for o in (-1, 0, 1):
        for row in range(CG):
            part, rem = divmod(row, G * FP)
            r, c = divmod(rem, FP)
            if c >= F:
                continue
            for s in range(G):
                t = G * o + r - s + ctr
                if 0 <= t < ksize:
                    tap[o + 1, row, s] = t
                    cin[o + 1, row, s] = part * F + c
    return tap, cin


def _band_maps_y(ksize, ctr):
    tap = np.full((3, KG, G), ksize, np.int32)
    cin = np.zeros((3, KG, G), np.int32)
    for o in (-1, 0, 1):
        for row in range(KG):
            r, kk = divmod(row, K)
            for s in range(G):
                t = G * o + r - s + ctr
                if 0 <= t < ksize:
                    tap[o + 1, row, s] = t
                    cin[o + 1, row, s] = 2 * F + kk
    return tap, cin


_W0_TAP, _W0_CIN = _band_maps(3, 1)
_W1X_TAP, _W1X_CIN = _band_maps(5, 2)
_W1Y_TAP, _W1Y_CIN = _band_maps_y(5, 2)

_WHX_CIN = np.full((CG, G), 2 * F + K, np.int32)
_WHY_CIN = np.full((KG, G), 2 * F + K, np.int32)
for _row in range(CG):
    _part, _rem = divmod(_row, G * FP)
    _r, _c = divmod(_rem, FP)
    if _c < F:
        _WHX_CIN[_row, _r] = _part * F + _c
for _row in range(KG):
    _r, _kk = divmod(_row, K)
    _WHY_CIN[_row, _r] = 2 * F + _kk


def _fused_kernel(prev_ref, cur_ref, next_ref,
                  w0b_ref, w1xb_ref, w1yb_ref, whx_ref, why_ref,
                  b0g_ref, b1g_ref, bhg_ref,
                  out_ref, *, n_groups):
    i = pl.program_id(0)
    f32 = jnp.float32
    xe = jnp.concatenate(
        [prev_ref[R - 2:, :], cur_ref[...], next_ref[:2, :]], axis=0)
    ge = i * R - 2 + jax.lax.broadcasted_iota(jnp.int32, (R + 4, 1), 0)
    xe = jnp.where((ge >= 0) & (ge < n_groups), xe, 0.0)
    xc = jnp.concatenate(
        [jnp.minimum(xe, 0.0), jnp.maximum(xe, 0.0)], axis=1)

    acc0 = jnp.broadcast_to(b0g_ref[...], (R + 2, KG)).astype(f32)
    for o in range(3):
        acc0 = acc0 + jnp.dot(xc[o:o + R + 2], w0b_ref[o],
                              preferred_element_type=f32)
    y0 = jnp.maximum(acc0, 0.0)
    y0 = jnp.where((ge[1:R + 3] >= 0) & (ge[1:R + 3] < n_groups), y0, 0.0)
    y0 = y0.astype(xe.dtype)

    acc1 = jnp.broadcast_to(b1g_ref[...], (R, KG)).astype(f32)
    for o in range(3):
        acc1 = acc1 + jnp.dot(xc[1 + o:1 + o + R], w1xb_ref[o],
                              preferred_element_type=f32)
        acc1 = acc1 + jnp.dot(y0[o:o + R], w1yb_ref[o],
                              preferred_element_type=f32)
    y1 = jnp.maximum(acc1, 0.0).astype(xe.dtype)

    z = (jnp.dot(xc[2:2 + R], whx_ref[...], preferred_element_type=f32)
         + jnp.dot(y1, why_ref[...], preferred_element_type=f32)
         + bhg_ref[...])
    out_ref[...] = jax.nn.softplus(z)


def kernel(inputs, W0, b0, W1, b1, Wh, bh):
    x = inputs[0]
    n, f = x.shape
    nb = n // T
    ng = n // G
    cdt = jnp.bfloat16

    xg = jnp.pad(x.astype(cdt), ((0, 0), (0, FP - f))).reshape(ng, G * FP)

    w0ext = jnp.pad(W0, ((0, 1), (0, 0), (0, 0)))
    w1ext = jnp.pad(W1, ((0, 1), (0, 0), (0, 0)))
    whext = jnp.pad(Wh, ((0, 1), (0, 0)))
    w0b = w0ext[_W0_TAP, _W0_CIN].reshape(3, CG, KG).astype(cdt)
    w1xb = w1ext[_W1X_TAP, _W1X_CIN].reshape(3, CG, KG).astype(cdt)
    w1yb = w1ext[_W1Y_TAP, _W1Y_CIN].reshape(3, KG, KG).astype(cdt)
    whx = whext[_WHX_CIN].reshape(CG, G * BINS).astype(cdt)
    why = whext[_WHY_CIN].reshape(KG, G * BINS).astype(cdt)
    b0g = jnp.tile(b0, G).reshape(1, KG)
    b1g = jnp.tile(b1, G).reshape(1, KG)
    bhg = jnp.tile(bh, G).reshape(1, G * BINS)

    full = lambda shape: pl.BlockSpec(shape, lambda i: (0,) * len(shape))
    out = pl.pallas_call(
        functools.partial(_fused_kernel, n_groups=ng),
        grid=(nb,),
        in_specs=[
            pl.BlockSpec((R, G * FP),
                         lambda i: (jnp.maximum(i - 1, 0), 0)),
            pl.BlockSpec((R, G * FP), lambda i: (i, 0)),
            pl.BlockSpec((R, G * FP),
                         lambda i: (jnp.minimum(i + 1, nb - 1), 0)),
            full((3, CG, KG)), full((3, CG, KG)), full((3, KG, KG)),
            full((CG, G * BINS)), full((KG, G * BINS)),
            full((1, KG)), full((1, KG)), full((1, G * BINS)),
        ],
        out_specs=pl.BlockSpec((R, G * BINS), lambda i: (i, 0)),
        out_shape=jax.ShapeDtypeStruct((ng, G * BINS), jnp.float32),
    )(xg, xg, xg, w0b, w1xb, w1yb, whx, why, b0g, b1g, bhg)
    return out.reshape(n, BINS)[None]


# speedup vs baseline: 1.0932x; 1.0638x over previous
import functools

import numpy as np

import jax
import jax.numpy as jnp
from jax.experimental import pallas as pl

F = 51
FP = 64
K = 128
BINS = 2
G = 2
T = 4096
R = T // G
CG = 2 * G * FP
KG = G * K


def _band_maps(ksize, ctr):
    tap = np.full((3, CG, G), ksize, np.int32)
    cin = np.zeros((3, CG, G), np.int32)
    for o in (-1, 0, 1):
        for row in range(CG):
            part, rem = divmod(row, G * FP)
            r, c = divmod(rem, FP)
            if c >= F:
                continue
            for s in range(G):
                t = G * o + r - s + ctr
                if 0 <= t < ksize:
                    tap[o + 1, row, s] = t
                    cin[o + 1, row, s] = part * F + c
    return tap, cin


def _band_maps_y(ksize, ctr):
    tap = np.full((3, KG, G), ksize, np.int32)
    cin = np.zeros((3, KG, G), np.int32)
    for o in (-1, 0, 1):
        for row in range(KG):
            r, kk = divmod(row, K)
            for s in range(G):
                t = G * o + r - s + ctr
                if 0 <= t < ksize:
                    tap[o + 1, row, s] = t
                    cin[o + 1, row, s] = 2 * F + kk
    return tap, cin


_W0_TAP, _W0_CIN = _band_maps(3, 1)
_W1X_TAP, _W1X_CIN = _band_maps(5, 2)
_W1Y_TAP, _W1Y_CIN = _band_maps_y(5, 2)

_WHX_CIN = np.full((CG, G), 2 * F + K, np.int32)
_WHY_CIN = np.full((KG, G), 2 * F + K, np.int32)
for _row in range(CG):
    _part, _rem = divmod(_row, G * FP)
    _r, _c = divmod(_rem, FP)
    if _c < F:
        _WHX_CIN[_row, _r] = _part * F + _c
for _row in range(KG):
    _r, _kk = divmod(_row, K)
    _WHY_CIN[_row, _r] = 2 * F + _kk


def _fused_kernel(prev_ref, cur_ref, next_ref,
                  w0b_ref, w1b_ref, whb_ref,
                  b0g_ref, b1g_ref, bhg_ref,
                  out_ref, *, n_groups):
    i = pl.program_id(0)
    f32 = jnp.float32
    xe = jnp.concatenate(
        [prev_ref[R - 2:, :], cur_ref[...], next_ref[:2, :]], axis=0)
    ge = i * R - 2 + jax.lax.broadcasted_iota(jnp.int32, (R + 4, 1), 0)
    xe = jnp.where((ge >= 0) & (ge < n_groups), xe, 0.0)
    xc = jnp.concatenate(
        [jnp.minimum(xe, 0.0), jnp.maximum(xe, 0.0)], axis=1)

    cat0 = jnp.concatenate(
        [xc[0:R + 2], xc[1:R + 3], xc[2:R + 4]], axis=1)
    acc0 = jnp.dot(cat0, w0b_ref[...],
                   preferred_element_type=f32) + b0g_ref[...]
    y0 = jnp.maximum(acc0, 0.0)
    y0 = jnp.where((ge[1:R + 3] >= 0) & (ge[1:R + 3] < n_groups), y0, 0.0)
    y0 = y0.astype(xe.dtype)

    cat1 = jnp.concatenate(
        [xc[1:R + 1], xc[2:R + 2], xc[3:R + 3],
         y0[0:R], y0[1:R + 1], y0[2:R + 2]], axis=1)
    acc1 = jnp.dot(cat1, w1b_ref[...],
                   preferred_element_type=f32) + b1g_ref[...]
    y1 = jnp.maximum(acc1, 0.0).astype(xe.dtype)

    cath = jnp.concatenate([xc[2:2 + R], y1], axis=1)
    z = jnp.dot(cath, whb_ref[...],
                preferred_element_type=f32) + bhg_ref[...]
    out_ref[...] = jax.nn.softplus(z)


def kernel(inputs, W0, b0, W1, b1, Wh, bh):
    x = inputs[0]
    n, f = x.shape
    nb = n // T
    ng = n // G
    cdt = jnp.bfloat16

    xg = jnp.pad(x.astype(cdt), ((0, 0), (0, FP - f))).reshape(ng, G * FP)

    w0ext = jnp.pad(W0, ((0, 1), (0, 0), (0, 0)))
    w1ext = jnp.pad(W1, ((0, 1), (0, 0), (0, 0)))
    whext = jnp.pad(Wh, ((0, 1), (0, 0)))
    w0b = w0ext[_W0_TAP, _W0_CIN].reshape(3 * CG, KG).astype(cdt)
    w1xb = w1ext[_W1X_TAP, _W1X_CIN].reshape(3 * CG, KG)
    w1yb = w1ext[_W1Y_TAP, _W1Y_CIN].reshape(3 * KG, KG)
    w1b = jnp.concatenate([w1xb, w1yb], axis=0).astype(cdt)
    whx = whext[_WHX_CIN].reshape(CG, G * BINS)
    why = whext[_WHY_CIN].reshape(KG, G * BINS)
    whb = jnp.concatenate([whx, why], axis=0).astype(cdt)
    b0g = jnp.tile(b0, G).reshape(1, KG)
    b1g = jnp.tile(b1, G).reshape(1, KG)
    bhg = jnp.tile(bh, G).reshape(1, G * BINS)

    full = lambda shape: pl.BlockSpec(shape, lambda i: (0,) * len(shape))
    out = pl.pallas_call(
        functools.partial(_fused_kernel, n_groups=ng),
        grid=(nb,),
        in_specs=[
            pl.BlockSpec((R, G * FP),
                         lambda i: (jnp.maximum(i - 1, 0), 0)),
            pl.BlockSpec((R, G * FP), lambda i: (i, 0)),
            pl.BlockSpec((R, G * FP),
                         lambda i: (jnp.minimum(i + 1, nb - 1), 0)),
            full((3 * CG, KG)), full((3 * CG + 3 * KG, KG)),
            full((CG + KG, G * BINS)),
            full((1, KG)), full((1, KG)), full((1, G * BINS)),
        ],
        out_specs=pl.BlockSpec((R, G * BINS), lambda i: (i, 0)),
        out_shape=jax.ShapeDtypeStruct((ng, G * BINS), jnp.float32),
    )(xg, xg, xg, w0b, w1b, whb, b0g, b1g, bhg)
    return out.reshape(n, BINS)[None]


# speedup vs baseline: 1.0949x; 1.0015x over previous
import functools

import numpy as np

import jax
import jax.numpy as jnp
from jax.experimental import pallas as pl
from jax.experimental.pallas import tpu as pltpu

F = 51
FP = 64
K = 128
BINS = 2
G = 2
T = 4096
R = T // G
CG = 2 * G * FP
KG = G * K


def _band_maps(ksize, ctr):
    tap = np.full((3, CG, G), ksize, np.int32)
    cin = np.zeros((3, CG, G), np.int32)
    for o in (-1, 0, 1):
        for row in range(CG):
            part, rem = divmod(row, G * FP)
            r, c = divmod(rem, FP)
            if c >= F:
                continue
            for s in range(G):
                t = G * o + r - s + ctr
                if 0 <= t < ksize:
                    tap[o + 1, row, s] = t
                    cin[o + 1, row, s] = part * F + c
    return tap, cin


def _band_maps_y(ksize, ctr):
    tap = np.full((3, KG, G), ksize, np.int32)
    cin = np.zeros((3, KG, G), np.int32)
    for o in (-1, 0, 1):
        for row in range(KG):
            r, kk = divmod(row, K)
            for s in range(G):
                t = G * o + r - s + ctr
                if 0 <= t < ksize:
                    tap[o + 1, row, s] = t
                    cin[o + 1, row, s] = 2 * F + kk
    return tap, cin


_W0_TAP, _W0_CIN = _band_maps(3, 1)
_W1X_TAP, _W1X_CIN = _band_maps(5, 2)
_W1Y_TAP, _W1Y_CIN = _band_maps_y(5, 2)

_WHX_CIN = np.full((CG, G), 2 * F + K, np.int32)
_WHY_CIN = np.full((KG, G), 2 * F + K, np.int32)
for _row in range(CG):
    _part, _rem = divmod(_row, G * FP)
    _r, _c = divmod(_rem, FP)
    if _c < F:
        _WHX_CIN[_row, _r] = _part * F + _c
for _row in range(KG):
    _r, _kk = divmod(_row, K)
    _WHY_CIN[_row, _r] = 2 * F + _kk


def _fused_kernel(prev_ref, cur_ref, next_ref,
                  w0b_ref, w1b_ref, whb_ref,
                  b0g_ref, b1g_ref, bhg_ref,
                  out_ref, *, n_groups):
    i = pl.program_id(0)
    f32 = jnp.float32
    xe = jnp.concatenate(
        [prev_ref[R - 2:, :], cur_ref[...], next_ref[:2, :]], axis=0)
    ge = i * R - 2 + jax.lax.broadcasted_iota(jnp.int32, (R + 4, 1), 0)
    xe = jnp.where((ge >= 0) & (ge < n_groups), xe, 0.0)
    xc = jnp.concatenate(
        [jnp.minimum(xe, 0.0), jnp.maximum(xe, 0.0)], axis=1)

    cat0 = jnp.concatenate(
        [xc[0:R + 2], xc[1:R + 3], xc[2:R + 4]], axis=1)
    acc0 = jnp.dot(cat0, w0b_ref[...],
                   preferred_element_type=f32) + b0g_ref[...]
    y0 = jnp.maximum(acc0, 0.0)
    y0 = jnp.where((ge[1:R + 3] >= 0) & (ge[1:R + 3] < n_groups), y0, 0.0)
    y0 = y0.astype(xe.dtype)

    cat1 = jnp.concatenate(
        [xc[1:R + 1], xc[2:R + 2], xc[3:R + 3],
         y0[0:R], y0[1:R + 1], y0[2:R + 2]], axis=1)
    acc1 = jnp.dot(cat1, w1b_ref[...],
                   preferred_element_type=f32) + b1g_ref[...]
    y1 = jnp.maximum(acc1, 0.0).astype(xe.dtype)

    cath = jnp.concatenate([xc[2:2 + R], y1], axis=1)
    z = jnp.dot(cath, whb_ref[...],
                preferred_element_type=f32) + bhg_ref[...]
    out_ref[...] = jax.nn.softplus(z)


def kernel(inputs, W0, b0, W1, b1, Wh, bh):
    x = inputs[0]
    n, f = x.shape
    nb = n // T
    ng = n // G
    cdt = jnp.bfloat16

    xg = jnp.pad(x.astype(cdt), ((0, 0), (0, FP - f))).reshape(ng, G * FP)

    w0ext = jnp.pad(W0, ((0, 1), (0, 0), (0, 0)))
    w1ext = jnp.pad(W1, ((0, 1), (0, 0), (0, 0)))
    whext = jnp.pad(Wh, ((0, 1), (0, 0)))
    w0b = w0ext[_W0_TAP, _W0_CIN].reshape(3 * CG, KG).astype(cdt)
    w1xb = w1ext[_W1X_TAP, _W1X_CIN].reshape(3 * CG, KG)
    w1yb = w1ext[_W1Y_TAP, _W1Y_CIN].reshape(3 * KG, KG)
    w1b = jnp.concatenate([w1xb, w1yb], axis=0).astype(cdt)
    whx = whext[_WHX_CIN].reshape(CG, G * BINS)
    why = whext[_WHY_CIN].reshape(KG, G * BINS)
    whb = jnp.concatenate([whx, why], axis=0).astype(cdt)
    b0g = jnp.tile(b0, G).reshape(1, KG)
    b1g = jnp.tile(b1, G).reshape(1, KG)
    bhg = jnp.tile(bh, G).reshape(1, G * BINS)

    full = lambda shape: pl.BlockSpec(shape, lambda i: (0,) * len(shape))
    out = pl.pallas_call(
        functools.partial(_fused_kernel, n_groups=ng),
        grid=(nb,),
        in_specs=[
            pl.BlockSpec((R, G * FP),
                         lambda i: (jnp.maximum(i - 1, 0), 0)),
            pl.BlockSpec((R, G * FP), lambda i: (i, 0)),
            pl.BlockSpec((R, G * FP),
                         lambda i: (jnp.minimum(i + 1, nb - 1), 0)),
            full((3 * CG, KG)), full((3 * CG + 3 * KG, KG)),
            full((CG + KG, G * BINS)),
            full((1, KG)), full((1, KG)), full((1, G * BINS)),
        ],
        out_specs=pl.BlockSpec((R, G * BINS), lambda i: (i, 0)),
        out_shape=jax.ShapeDtypeStruct((ng, G * BINS), jnp.float32),
        compiler_params=pltpu.CompilerParams(
            dimension_semantics=("parallel",)),
    )(xg, xg, xg, w0b, w1b, whb, b0g, b1g, bhg)
    return out.reshape(n, BINS)[None]
